# baseline (speedup 1.0000x reference)
import functools

import jax
import jax.numpy as jnp
from jax import lax
from jax.experimental import pallas as pl
from jax.experimental.pallas import tpu as pltpu
from jax.experimental.pallas import tpu_sc as plsc

N = 10000
E = 320000
D = 128

NC = 2
NS = 16
L = 16
NW = NC * NS

NP = 10240
SPT = NP // NS
EPT_DEG = E // NS
RPT = NP // NS
EPC = E // NW
CH = 128
NCH = 79
EPP = NCH * CH
NPAD = EPP - EPC

_MESH = plsc.VectorSubcoreMesh(
    core_axis_name="c", subcore_axis_name="s", num_cores=NC, num_subcores=NS)


def _rsqrt16(x):
    xi = plsc.bitcast(x, jnp.int32)
    yi = jnp.int32(0x5F3759DF) - (xi >> 1)
    y = plsc.bitcast(yi, jnp.float32)
    for _ in range(3):
        y = y * (1.5 - 0.5 * x * y * y)
    return y


def _dis_body(row_hbm, dis_hbm, idx_v, hist_v, dis_v, merged_v, shist):
    cid = lax.axis_index("c")
    sid = lax.axis_index("s")

    def zero(i, c):
        hist_v[pl.ds(i * L, L)] = jnp.zeros((L,), jnp.float32)
        return c
    lax.fori_loop(0, NP // L, zero, 0)

    pltpu.sync_copy(row_hbm.at[pl.ds(sid * EPT_DEG, EPT_DEG)], idx_v)

    ones = jnp.ones((L,), jnp.float32)

    def hist(i, c):
        idx = idx_v[pl.ds(i * L, L)]
        plsc.addupdate_scatter(hist_v, [idx], ones)
        return c
    lax.fori_loop(0, EPT_DEG // L, hist, 0)

    pltpu.sync_copy(hist_v, shist.at[sid])
    plsc.subcore_barrier()
    pltpu.sync_copy(shist.at[:, pl.ds(sid * SPT, SPT)], merged_v)

    def merge(j, c):
        acc = jnp.ones((L,), jnp.float32)
        for s in range(NS):
            acc = acc + merged_v[s, pl.ds(j * L, L)]
        dis_v[pl.ds(j * L, L)] = _rsqrt16(acc)
        return c
    lax.fori_loop(0, SPT // L, merge, 0)

    @pl.when(cid == 0)
    def _():
        pltpu.sync_copy(dis_v, dis_hbm.at[pl.ds(sid * SPT, SPT)])


_dis_call = functools.partial(
    pl.kernel,
    out_type=jax.ShapeDtypeStruct((NP,), jnp.float32),
    mesh=_MESH,
    scratch_types=[
        pltpu.VMEM((EPT_DEG,), jnp.int32),
        pltpu.VMEM((NP,), jnp.float32),
        pltpu.VMEM((SPT,), jnp.float32),
        pltpu.VMEM((NS, SPT), jnp.float32),
        pltpu.VMEM_SHARED((NS, NP), jnp.float32),
    ],
    compiler_params=pltpu.CompilerParams(needs_layout_passes=False),
)(_dis_body)


def _scatter_body(hs_hbm, row_hbm, col_hbm, z_hbm, acc_hbm,
                  cidx_v, ridx_v, gbuf_v, acc_sh, csem, rsem, gsem, ssem):
    cid = lax.axis_index("c")
    sid = lax.axis_index("s")
    wid = cid * NS + sid

    pltpu.sync_copy(z_hbm.at[pl.ds(sid * RPT, RPT)],
                    acc_sh.at[pl.ds(sid * RPT, RPT)])
    plsc.subcore_barrier()

    def load_idx(k):
        s = k % 3
        pltpu.async_copy(col_hbm.at[wid, k], cidx_v.at[s], csem.at[s])
        pltpu.async_copy(row_hbm.at[wid, k], ridx_v.at[s], rsem.at[s])

    def wait_cidx(k):
        s = k % 3
        pltpu.make_async_copy(
            col_hbm.at[wid, k], cidx_v.at[s], csem.at[s]).wait()

    def wait_ridx(k):
        s = k % 3
        pltpu.make_async_copy(
            row_hbm.at[wid, k], ridx_v.at[s], rsem.at[s]).wait()

    def gather(k):
        pltpu.async_copy(
            hs_hbm.at[cidx_v.at[k % 3]], gbuf_v.at[k % 2], gsem.at[k % 2])

    def wait_gather(k):
        pltpu.make_async_copy(
            hs_hbm.at[cidx_v.at[k % 3]], gbuf_v.at[k % 2],
            gsem.at[k % 2]).wait()

    def scatter(k):
        pltpu.async_copy(gbuf_v.at[k % 2], acc_sh.at[ridx_v.at[k % 3]],
                         ssem.at[k % 2], add=True)

    def wait_scatter(k):
        pltpu.make_async_copy(
            gbuf_v.at[k % 2], acc_sh.at[ridx_v.at[k % 3]],
            ssem.at[k % 2]).wait()

    load_idx(0)
    load_idx(1)
    wait_cidx(0)
    gather(0)

    def body(k, c):
        wait_gather(k)
        wait_ridx(k)
        scatter(k)

        @pl.when(k + 1 < NCH)
        def _():
            wait_cidx(k + 1)
            @pl.when(k >= 1)
            def _():
                wait_scatter(k - 1)
            gather(k + 1)

        @pl.when(k + 2 < NCH)
        def _():
            load_idx(k + 2)
        return c
    lax.fori_loop(0, NCH, body, 0)
    wait_scatter(NCH - 2)
    wait_scatter(NCH - 1)

    plsc.subcore_barrier()
    pltpu.sync_copy(acc_sh.at[pl.ds(sid * RPT, RPT)],
                    acc_hbm.at[cid, pl.ds(sid * RPT, RPT)])


_scatter_call = functools.partial(
    pl.kernel,
    out_type=jax.ShapeDtypeStruct((NC, NP, D), jnp.float32),
    mesh=_MESH,
    scratch_types=[
        pltpu.VMEM((3, CH), jnp.int32),
        pltpu.VMEM((3, CH), jnp.int32),
        pltpu.VMEM((2, CH, D), jnp.float32),
        pltpu.VMEM_SHARED((NP, D), jnp.float32),
        pltpu.SemaphoreType.DMA((3,)),
        pltpu.SemaphoreType.DMA((3,)),
        pltpu.SemaphoreType.DMA((2,)),
        pltpu.SemaphoreType.DMA((2,)),
    ],
    compiler_params=pltpu.CompilerParams(needs_layout_passes=False),
)(_scatter_body)


BM = 400


def _matmul_body(x_ref, d_ref, w_ref, o_ref):
    o_ref[...] = jnp.dot(x_ref[...] * d_ref[...], w_ref[...],
                         preferred_element_type=jnp.float32)


def _matmul_call(x, d2, w):
    return pl.pallas_call(
        _matmul_body,
        grid=(N // BM,),
        in_specs=[
            pl.BlockSpec((BM, D), lambda i: (i, 0)),
            pl.BlockSpec((BM, 1), lambda i: (i, 0)),
            pl.BlockSpec((D, D), lambda i: (0, 0)),
        ],
        out_specs=pl.BlockSpec((BM, D), lambda i: (i, 0)),
        out_shape=jax.ShapeDtypeStruct((N, D), jnp.float32),
    )(x, d2, w)


def _merge_body(a0_ref, a1_ref, hs_ref, d_ref, b_ref, o_ref):
    o_ref[...] = (d_ref[...] * (a0_ref[...] + a1_ref[...] + hs_ref[...])
                  + b_ref[...])


def _merge_call(a0, a1, hs, d2, b2):
    return pl.pallas_call(
        _merge_body,
        grid=(N // BM,),
        in_specs=[
            pl.BlockSpec((BM, D), lambda i: (i, 0)),
            pl.BlockSpec((BM, D), lambda i: (i, 0)),
            pl.BlockSpec((BM, D), lambda i: (i, 0)),
            pl.BlockSpec((BM, 1), lambda i: (i, 0)),
            pl.BlockSpec((1, D), lambda i: (0, 0)),
        ],
        out_specs=pl.BlockSpec((BM, D), lambda i: (i, 0)),
        out_shape=jax.ShapeDtypeStruct((N, D), jnp.float32),
    )(a0, a1, hs, d2, b2)


def kernel(nodes_ft, adj_list, W, b):
    row = adj_list[0]
    col = adj_list[1]
    dis_pad = _dis_call(row)
    d2 = dis_pad[:N, None]
    hs = _matmul_call(nodes_ft, d2, W)
    z = jnp.zeros((NP, D), jnp.float32)
    pad_col = jnp.broadcast_to((jnp.arange(NPAD, dtype=jnp.int32) * 89) % N,
                               (NW, NPAD))
    pad_row = jnp.broadcast_to(N + jnp.arange(NPAD, dtype=jnp.int32),
                               (NW, NPAD))
    row3 = jnp.concatenate([row.reshape(NW, EPC), pad_row],
                           axis=1).reshape(NW, NCH, CH)
    col3 = jnp.concatenate([col.reshape(NW, EPC), pad_col],
                           axis=1).reshape(NW, NCH, CH)
    acc = _scatter_call(hs, row3, col3, z)
    out = _merge_call(acc[0, :N], acc[1, :N], hs, d2, b.reshape(1, D))
    return out


# speedup vs baseline: 1.1817x; 1.1817x over previous
import functools

import jax
import jax.numpy as jnp
from jax import lax
from jax.experimental import pallas as pl
from jax.experimental.pallas import tpu as pltpu
from jax.experimental.pallas import tpu_sc as plsc

N = 10000
E = 320000
D = 128

NC = 2
NS = 16
L = 16
NW = NC * NS

NP = 10240
SPT = NP // NS
EPT_DEG = E // NS
NPA = 10112
RPT = NPA // NS
EPC = E // NW
CH = 120
NCH = 84
EPP = NCH * CH
NPAD = EPP - EPC

_MESH = plsc.VectorSubcoreMesh(
    core_axis_name="c", subcore_axis_name="s", num_cores=NC, num_subcores=NS)


def _rsqrt16(x):
    xi = plsc.bitcast(x, jnp.int32)
    yi = jnp.int32(0x5F3759DF) - (xi >> 1)
    y = plsc.bitcast(yi, jnp.float32)
    for _ in range(3):
        y = y * (1.5 - 0.5 * x * y * y)
    return y


def _dis_body(row_hbm, dis_hbm, idx_v, hist_v, dis_v, merged_v, shist):
    cid = lax.axis_index("c")
    sid = lax.axis_index("s")

    def zero(i, c):
        hist_v[pl.ds(i * L, L)] = jnp.zeros((L,), jnp.float32)
        return c
    lax.fori_loop(0, NP // L, zero, 0)

    pltpu.sync_copy(row_hbm.at[pl.ds(sid * EPT_DEG, EPT_DEG)], idx_v)

    ones = jnp.ones((L,), jnp.float32)

    def hist(i, c):
        idx = idx_v[pl.ds(i * L, L)]
        plsc.addupdate_scatter(hist_v, [idx], ones)
        return c
    lax.fori_loop(0, EPT_DEG // L, hist, 0)

    pltpu.sync_copy(hist_v, shist.at[sid])
    plsc.subcore_barrier()
    pltpu.sync_copy(shist.at[:, pl.ds(sid * SPT, SPT)], merged_v)

    def merge(j, c):
        acc = jnp.ones((L,), jnp.float32)
        for s in range(NS):
            acc = acc + merged_v[s, pl.ds(j * L, L)]
        dis_v[pl.ds(j * L, L)] = _rsqrt16(acc)
        return c
    lax.fori_loop(0, SPT // L, merge, 0)

    @pl.when(cid == 0)
    def _():
        pltpu.sync_copy(dis_v, dis_hbm.at[pl.ds(sid * SPT, SPT)])


_dis_call = functools.partial(
    pl.kernel,
    out_type=jax.ShapeDtypeStruct((NP,), jnp.float32),
    mesh=_MESH,
    scratch_types=[
        pltpu.VMEM((EPT_DEG,), jnp.int32),
        pltpu.VMEM((NP,), jnp.float32),
        pltpu.VMEM((SPT,), jnp.float32),
        pltpu.VMEM((NS, SPT), jnp.float32),
        pltpu.VMEM_SHARED((NS, NP), jnp.float32),
    ],
    compiler_params=pltpu.CompilerParams(needs_layout_passes=False),
)(_dis_body)


def _scatter_body(hs_hbm, row_hbm, col_hbm, z_hbm, acc_hbm,
                  idx_v, gbuf_v, acc_sh, csem, rsem, gsem):
    cid = lax.axis_index("c")
    sid = lax.axis_index("s")
    wid = cid * NS + sid

    pltpu.sync_copy(z_hbm.at[pl.ds(sid * RPT, RPT)],
                    acc_sh.at[pl.ds(sid * RPT, RPT)])
    plsc.subcore_barrier()


    def load_idx(k):
        s = k % 4
        pltpu.async_copy(col_hbm.at[wid, k], idx_v.at[s], csem.at[s])
        pltpu.async_copy(row_hbm.at[wid, k], idx_v.at[4 + s], rsem.at[s])

    def wait_cidx(k):
        s = k % 4
        pltpu.make_async_copy(
            col_hbm.at[wid, k], idx_v.at[s], csem.at[s]).wait()

    def wait_ridx(k):
        s = k % 4
        pltpu.make_async_copy(
            row_hbm.at[wid, k], idx_v.at[4 + s], rsem.at[s]).wait()

    def gather(k):
        pltpu.async_copy(
            hs_hbm.at[idx_v.at[k % 4]], gbuf_v.at[k % 3], gsem.at[k % 3])

    def wait_gather(k):
        pltpu.make_async_copy(
            hs_hbm.at[idx_v.at[k % 4]], gbuf_v.at[k % 3],
            gsem.at[k % 3]).wait()

    load_idx(0)
    load_idx(1)
    load_idx(2)
    wait_cidx(0)
    gather(0)
    wait_cidx(1)
    gather(1)

    def body(k, c):
        @pl.when(k + 2 < NCH)
        def _():
            wait_cidx(k + 2)
            gather(k + 2)

        @pl.when(k + 3 < NCH)
        def _():
            load_idx(k + 3)

        wait_gather(k)
        wait_ridx(k)
        pltpu.sync_copy(gbuf_v.at[k % 3], acc_sh.at[idx_v.at[4 + k % 4]],
                        add=True)
        return c
    lax.fori_loop(0, NCH, body, 0)

    plsc.subcore_barrier()
    pltpu.sync_copy(acc_sh.at[pl.ds(sid * RPT, RPT)],
                    acc_hbm.at[cid, pl.ds(sid * RPT, RPT)])


_scatter_call = functools.partial(
    pl.kernel,
    out_type=jax.ShapeDtypeStruct((NC, NPA, D), jnp.float32),
    mesh=_MESH,
    scratch_types=[
        pltpu.VMEM((8, CH), jnp.int32),
        pltpu.VMEM((3, CH, D), jnp.float32),
        pltpu.VMEM_SHARED((NPA, D), jnp.float32),
        pltpu.SemaphoreType.DMA((4,)),
        pltpu.SemaphoreType.DMA((4,)),
        pltpu.SemaphoreType.DMA((3,)),
    ],
    compiler_params=pltpu.CompilerParams(needs_layout_passes=False),
)(_scatter_body)


BM = 400


def _matmul_body(x_ref, d_ref, w_ref, o_ref):
    o_ref[...] = jnp.dot(x_ref[...] * d_ref[...], w_ref[...],
                         preferred_element_type=jnp.float32)


def _matmul_call(x, d2, w):
    return pl.pallas_call(
        _matmul_body,
        grid=(N // BM,),
        in_specs=[
            pl.BlockSpec((BM, D), lambda i: (i, 0)),
            pl.BlockSpec((BM, 1), lambda i: (i, 0)),
            pl.BlockSpec((D, D), lambda i: (0, 0)),
        ],
        out_specs=pl.BlockSpec((BM, D), lambda i: (i, 0)),
        out_shape=jax.ShapeDtypeStruct((N, D), jnp.float32),
    )(x, d2, w)


def _merge_body(a0_ref, a1_ref, hs_ref, d_ref, b_ref, o_ref):
    o_ref[...] = (d_ref[...] * (a0_ref[...] + a1_ref[...] + hs_ref[...])
                  + b_ref[...])


def _merge_call(a0, a1, hs, d2, b2):
    return pl.pallas_call(
        _merge_body,
        grid=(N // BM,),
        in_specs=[
            pl.BlockSpec((BM, D), lambda i: (i, 0)),
            pl.BlockSpec((BM, D), lambda i: (i, 0)),
            pl.BlockSpec((BM, D), lambda i: (i, 0)),
            pl.BlockSpec((BM, 1), lambda i: (i, 0)),
            pl.BlockSpec((1, D), lambda i: (0, 0)),
        ],
        out_specs=pl.BlockSpec((BM, D), lambda i: (i, 0)),
        out_shape=jax.ShapeDtypeStruct((N, D), jnp.float32),
    )(a0, a1, hs, d2, b2)


def kernel(nodes_ft, adj_list, W, b):
    row = adj_list[0]
    col = adj_list[1]
    dis_pad = _dis_call(row)
    d2 = dis_pad[:N, None]
    hs = _matmul_call(nodes_ft, d2, W)
    z = jnp.zeros((NPA, D), jnp.float32)
    pad_col = jnp.broadcast_to((jnp.arange(NPAD, dtype=jnp.int32) * 89) % N,
                               (NW, NPAD))
    pad_row = jnp.broadcast_to(
        N + jnp.arange(NPAD, dtype=jnp.int32) % (NPA - N), (NW, NPAD))
    row3 = jnp.concatenate([row.reshape(NW, EPC), pad_row],
                           axis=1).reshape(NW, NCH, CH)
    col3 = jnp.concatenate([col.reshape(NW, EPC), pad_col],
                           axis=1).reshape(NW, NCH, CH)
    acc = _scatter_call(hs, row3, col3, z)
    out = _merge_call(acc[0, :N], acc[1, :N], hs, d2, b.reshape(1, D))
    return out


# speedup vs baseline: 1.1859x; 1.0035x over previous
import functools

import jax
import jax.numpy as jnp
from jax import lax
from jax.experimental import pallas as pl
from jax.experimental.pallas import tpu as pltpu
from jax.experimental.pallas import tpu_sc as plsc

N = 10000
E = 320000
D = 128

NC = 2
NS = 16
L = 16
NW = NC * NS

NP = 10240
SPT = NP // NS
EPT_DEG = E // NS
NPA = 10112
RPT = NPA // NS
EPC = E // NW
CH = 120
NCH = 84
EPP = NCH * CH
NPAD = EPP - EPC

_MESH = plsc.VectorSubcoreMesh(
    core_axis_name="c", subcore_axis_name="s", num_cores=NC, num_subcores=NS)


def _rsqrt16(x):
    xi = plsc.bitcast(x, jnp.int32)
    yi = jnp.int32(0x5F3759DF) - (xi >> 1)
    y = plsc.bitcast(yi, jnp.float32)
    for _ in range(3):
        y = y * (1.5 - 0.5 * x * y * y)
    return y


def _dis_body(row_hbm, dis_hbm, idx_v, hist_v, dis_v, merged_v, shist):
    cid = lax.axis_index("c")
    sid = lax.axis_index("s")

    def zero(i, c):
        hist_v[pl.ds(i * L, L)] = jnp.zeros((L,), jnp.float32)
        return c
    lax.fori_loop(0, NP // L, zero, 0)

    pltpu.sync_copy(row_hbm.at[pl.ds(sid * EPT_DEG, EPT_DEG)], idx_v)

    ones = jnp.ones((L,), jnp.float32)

    def hist(i, c):
        for u in range(10):
            idx = idx_v[pl.ds(i * (10 * L) + u * L, L)]
            plsc.addupdate_scatter(hist_v, [idx], ones)
        return c
    lax.fori_loop(0, EPT_DEG // (10 * L), hist, 0)

    pltpu.sync_copy(hist_v, shist.at[sid])
    plsc.subcore_barrier()
    pltpu.sync_copy(shist.at[:, pl.ds(sid * SPT, SPT)], merged_v)

    def merge(j, c):
        acc = jnp.ones((L,), jnp.float32)
        for s in range(NS):
            acc = acc + merged_v[s, pl.ds(j * L, L)]
        dis_v[pl.ds(j * L, L)] = _rsqrt16(acc)
        return c
    lax.fori_loop(0, SPT // L, merge, 0)

    @pl.when(cid == 0)
    def _():
        pltpu.sync_copy(dis_v, dis_hbm.at[pl.ds(sid * SPT, SPT)])


_dis_call = functools.partial(
    pl.kernel,
    out_type=jax.ShapeDtypeStruct((NP,), jnp.float32),
    mesh=_MESH,
    scratch_types=[
        pltpu.VMEM((EPT_DEG,), jnp.int32),
        pltpu.VMEM((NP,), jnp.float32),
        pltpu.VMEM((SPT,), jnp.float32),
        pltpu.VMEM((NS, SPT), jnp.float32),
        pltpu.VMEM_SHARED((NS, NP), jnp.float32),
    ],
    compiler_params=pltpu.CompilerParams(needs_layout_passes=False),
)(_dis_body)


def _scatter_body(hs_hbm, row_hbm, col_hbm, z_hbm, acc_hbm,
                  idx_v, gbuf_v, acc_sh, csem, rsem, gsem):
    cid = lax.axis_index("c")
    sid = lax.axis_index("s")
    wid = cid * NS + sid

    pltpu.sync_copy(z_hbm.at[pl.ds(sid * RPT, RPT)],
                    acc_sh.at[pl.ds(sid * RPT, RPT)])
    plsc.subcore_barrier()


    def load_idx(k):
        s = k % 4
        pltpu.async_copy(col_hbm.at[wid, k], idx_v.at[s], csem.at[s])
        pltpu.async_copy(row_hbm.at[wid, k], idx_v.at[4 + s], rsem.at[s])

    def wait_cidx(k):
        s = k % 4
        pltpu.make_async_copy(
            col_hbm.at[wid, k], idx_v.at[s], csem.at[s]).wait()

    def wait_ridx(k):
        s = k % 4
        pltpu.make_async_copy(
            row_hbm.at[wid, k], idx_v.at[4 + s], rsem.at[s]).wait()

    def gather(k):
        pltpu.async_copy(
            hs_hbm.at[idx_v.at[k % 4]], gbuf_v.at[k % 3], gsem.at[k % 3])

    def wait_gather(k):
        pltpu.make_async_copy(
            hs_hbm.at[idx_v.at[k % 4]], gbuf_v.at[k % 3],
            gsem.at[k % 3]).wait()

    load_idx(0)
    load_idx(1)
    load_idx(2)
    wait_cidx(0)
    gather(0)
    wait_cidx(1)
    gather(1)

    def body(k, c):
        @pl.when(k + 2 < NCH)
        def _():
            wait_cidx(k + 2)
            gather(k + 2)

        @pl.when(k + 3 < NCH)
        def _():
            load_idx(k + 3)

        wait_gather(k)
        wait_ridx(k)
        pltpu.sync_copy(gbuf_v.at[k % 3], acc_sh.at[idx_v.at[4 + k % 4]],
                        add=True)
        return c
    lax.fori_loop(0, NCH, body, 0)

    plsc.subcore_barrier()
    pltpu.sync_copy(acc_sh.at[pl.ds(sid * RPT, RPT)],
                    acc_hbm.at[cid, pl.ds(sid * RPT, RPT)])


_scatter_call = functools.partial(
    pl.kernel,
    out_type=jax.ShapeDtypeStruct((NC, NPA, D), jnp.float32),
    mesh=_MESH,
    scratch_types=[
        pltpu.VMEM((8, CH), jnp.int32),
        pltpu.VMEM((3, CH, D), jnp.float32),
        pltpu.VMEM_SHARED((NPA, D), jnp.float32),
        pltpu.SemaphoreType.DMA((4,)),
        pltpu.SemaphoreType.DMA((4,)),
        pltpu.SemaphoreType.DMA((3,)),
    ],
    compiler_params=pltpu.CompilerParams(needs_layout_passes=False),
)(_scatter_body)


BM = 400


def _matmul_body(x_ref, d_ref, w_ref, o_ref):
    o_ref[...] = jnp.dot(x_ref[...] * d_ref[...], w_ref[...],
                         preferred_element_type=jnp.float32)


def _matmul_call(x, d2, w):
    return pl.pallas_call(
        _matmul_body,
        grid=(N // BM,),
        in_specs=[
            pl.BlockSpec((BM, D), lambda i: (i, 0)),
            pl.BlockSpec((BM, 1), lambda i: (i, 0)),
            pl.BlockSpec((D, D), lambda i: (0, 0)),
        ],
        out_specs=pl.BlockSpec((BM, D), lambda i: (i, 0)),
        out_shape=jax.ShapeDtypeStruct((N, D), jnp.float32),
    )(x, d2, w)


def _merge_body(a0_ref, a1_ref, hs_ref, d_ref, b_ref, o_ref):
    o_ref[...] = (d_ref[...] * (a0_ref[...] + a1_ref[...] + hs_ref[...])
                  + b_ref[...])


def _merge_call(a0, a1, hs, d2, b2):
    return pl.pallas_call(
        _merge_body,
        grid=(N // BM,),
        in_specs=[
            pl.BlockSpec((BM, D), lambda i: (i, 0)),
            pl.BlockSpec((BM, D), lambda i: (i, 0)),
            pl.BlockSpec((BM, D), lambda i: (i, 0)),
            pl.BlockSpec((BM, 1), lambda i: (i, 0)),
            pl.BlockSpec((1, D), lambda i: (0, 0)),
        ],
        out_specs=pl.BlockSpec((BM, D), lambda i: (i, 0)),
        out_shape=jax.ShapeDtypeStruct((N, D), jnp.float32),
    )(a0, a1, hs, d2, b2)


def kernel(nodes_ft, adj_list, W, b):
    row = adj_list[0]
    col = adj_list[1]
    dis_pad = _dis_call(row)
    d2 = dis_pad[:N, None]
    hs = _matmul_call(nodes_ft, d2, W)
    z = jnp.zeros((NPA, D), jnp.float32)
    pad_col = jnp.broadcast_to((jnp.arange(NPAD, dtype=jnp.int32) * 89) % N,
                               (NW, NPAD))
    pad_row = jnp.broadcast_to(
        N + jnp.arange(NPAD, dtype=jnp.int32) % (NPA - N), (NW, NPAD))
    row3 = jnp.concatenate([row.reshape(NW, EPC), pad_row],
                           axis=1).reshape(NW, NCH, CH)
    col3 = jnp.concatenate([col.reshape(NW, EPC), pad_col],
                           axis=1).reshape(NW, NCH, CH)
    acc = _scatter_call(hs, row3, col3, z)
    out = _merge_call(acc[0, :N], acc[1, :N], hs, d2, b.reshape(1, D))
    return out


# speedup vs baseline: 1.2211x; 1.0297x over previous
import functools

import jax
import jax.numpy as jnp
from jax import lax
from jax.experimental import pallas as pl
from jax.experimental.pallas import tpu as pltpu
from jax.experimental.pallas import tpu_sc as plsc

N = 10000
E = 320000
D = 128

NC = 2
NS = 16
L = 16
NW = NC * NS

NP = 10240
SPT = NP // NS
EPT_DEG = E // NS
NPA = 10112
RPT = NPA // NS
EPC = E // NW
CH = 120
NCH = 84
EPP = NCH * CH
NPAD = EPP - EPC

_MESH = plsc.VectorSubcoreMesh(
    core_axis_name="c", subcore_axis_name="s", num_cores=NC, num_subcores=NS)


def _rsqrt16(x):
    xi = plsc.bitcast(x, jnp.int32)
    yi = jnp.int32(0x5F3759DF) - (xi >> 1)
    y = plsc.bitcast(yi, jnp.float32)
    for _ in range(3):
        y = y * (1.5 - 0.5 * x * y * y)
    return y


def _dis_body(row_hbm, dis_hbm, idx_v, hist_v, dis_v, merged_v, shist):
    cid = lax.axis_index("c")
    sid = lax.axis_index("s")

    def zero(i, c):
        hist_v[pl.ds(i * L, L)] = jnp.zeros((L,), jnp.float32)
        return c
    lax.fori_loop(0, NP // L, zero, 0)

    pltpu.sync_copy(row_hbm.at[pl.ds(sid * EPT_DEG, EPT_DEG)], idx_v)

    ones = jnp.ones((L,), jnp.float32)

    def hist(i, c):
        for u in range(10):
            idx = idx_v[pl.ds(i * (10 * L) + u * L, L)]
            plsc.addupdate_scatter(hist_v, [idx], ones)
        return c
    lax.fori_loop(0, EPT_DEG // (10 * L), hist, 0)

    pltpu.sync_copy(hist_v, shist.at[sid])
    plsc.subcore_barrier()
    pltpu.sync_copy(shist.at[:, pl.ds(sid * SPT, SPT)], merged_v)

    def merge(j, c):
        acc = jnp.ones((L,), jnp.float32)
        for s in range(NS):
            acc = acc + merged_v[s, pl.ds(j * L, L)]
        dis_v[pl.ds(j * L, L)] = _rsqrt16(acc)
        return c
    lax.fori_loop(0, SPT // L, merge, 0)

    @pl.when(cid == 0)
    def _():
        pltpu.sync_copy(dis_v, dis_hbm.at[pl.ds(sid * SPT, SPT)])


_dis_call = functools.partial(
    pl.kernel,
    out_type=jax.ShapeDtypeStruct((NP,), jnp.float32),
    mesh=_MESH,
    scratch_types=[
        pltpu.VMEM((EPT_DEG,), jnp.int32),
        pltpu.VMEM((NP,), jnp.float32),
        pltpu.VMEM((SPT,), jnp.float32),
        pltpu.VMEM((NS, SPT), jnp.float32),
        pltpu.VMEM_SHARED((NS, NP), jnp.float32),
    ],
    compiler_params=pltpu.CompilerParams(needs_layout_passes=False),
)(_dis_body)


def _scatter_body(hs_hbm, row_hbm, col_hbm, acc_hbm,
                  idx_v, gbuf_v, acc_sh, csem, rsem, gsem):
    cid = lax.axis_index("c")
    sid = lax.axis_index("s")
    wid = cid * NS + sid

    def zrow(r, c):
        for u in range(D // L):
            gbuf_v[0, r, pl.ds(u * L, L)] = jnp.zeros((L,), jnp.float32)
        return c
    lax.fori_loop(0, CH, zrow, 0)
    for t in range(RPT // CH):
        pltpu.sync_copy(gbuf_v.at[0],
                        acc_sh.at[pl.ds(sid * RPT + t * CH, CH)])
    rem = RPT % CH
    if rem:
        pltpu.sync_copy(gbuf_v.at[0, pl.ds(0, rem)],
                        acc_sh.at[pl.ds(sid * RPT + RPT - rem, rem)])
    plsc.subcore_barrier()


    def load_idx(k):
        s = k % 4
        pltpu.async_copy(col_hbm.at[wid, k], idx_v.at[s], csem.at[s])
        pltpu.async_copy(row_hbm.at[wid, k], idx_v.at[4 + s], rsem.at[s])

    def wait_cidx(k):
        s = k % 4
        pltpu.make_async_copy(
            col_hbm.at[wid, k], idx_v.at[s], csem.at[s]).wait()

    def wait_ridx(k):
        s = k % 4
        pltpu.make_async_copy(
            row_hbm.at[wid, k], idx_v.at[4 + s], rsem.at[s]).wait()

    def gather(k):
        pltpu.async_copy(
            hs_hbm.at[idx_v.at[k % 4]], gbuf_v.at[k % 3], gsem.at[k % 3])

    def wait_gather(k):
        pltpu.make_async_copy(
            hs_hbm.at[idx_v.at[k % 4]], gbuf_v.at[k % 3],
            gsem.at[k % 3]).wait()

    load_idx(0)
    load_idx(1)
    load_idx(2)
    wait_cidx(0)
    gather(0)
    wait_cidx(1)
    gather(1)

    def body(k, c):
        @pl.when(k + 2 < NCH)
        def _():
            wait_cidx(k + 2)
            gather(k + 2)

        @pl.when(k + 3 < NCH)
        def _():
            load_idx(k + 3)

        wait_gather(k)
        wait_ridx(k)
        pltpu.sync_copy(gbuf_v.at[k % 3], acc_sh.at[idx_v.at[4 + k % 4]],
                        add=True)
        return c
    lax.fori_loop(0, NCH, body, 0)

    plsc.subcore_barrier()
    pltpu.sync_copy(acc_sh.at[pl.ds(sid * RPT, RPT)],
                    acc_hbm.at[cid, pl.ds(sid * RPT, RPT)])


_scatter_call = functools.partial(
    pl.kernel,
    out_type=jax.ShapeDtypeStruct((NC, NPA, D), jnp.float32),
    mesh=_MESH,
    scratch_types=[
        pltpu.VMEM((8, CH), jnp.int32),
        pltpu.VMEM((3, CH, D), jnp.float32),
        pltpu.VMEM_SHARED((NPA, D), jnp.float32),
        pltpu.SemaphoreType.DMA((4,)),
        pltpu.SemaphoreType.DMA((4,)),
        pltpu.SemaphoreType.DMA((3,)),
    ],
    compiler_params=pltpu.CompilerParams(needs_layout_passes=False),
)(_scatter_body)


BM = 400


def _matmul_body(x_ref, d_ref, w_ref, o_ref):
    o_ref[...] = jnp.dot(x_ref[...] * d_ref[...], w_ref[...],
                         preferred_element_type=jnp.float32)


def _matmul_call(x, d2, w):
    return pl.pallas_call(
        _matmul_body,
        grid=(N // BM,),
        in_specs=[
            pl.BlockSpec((BM, D), lambda i: (i, 0)),
            pl.BlockSpec((BM, 1), lambda i: (i, 0)),
            pl.BlockSpec((D, D), lambda i: (0, 0)),
        ],
        out_specs=pl.BlockSpec((BM, D), lambda i: (i, 0)),
        out_shape=jax.ShapeDtypeStruct((N, D), jnp.float32),
    )(x, d2, w)


def _merge_body(a0_ref, a1_ref, hs_ref, d_ref, b_ref, o_ref):
    o_ref[...] = (d_ref[...] * (a0_ref[...] + a1_ref[...] + hs_ref[...])
                  + b_ref[...])


def _merge_call(a0, a1, hs, d2, b2):
    return pl.pallas_call(
        _merge_body,
        grid=(N // BM,),
        in_specs=[
            pl.BlockSpec((BM, D), lambda i: (i, 0)),
            pl.BlockSpec((BM, D), lambda i: (i, 0)),
            pl.BlockSpec((BM, D), lambda i: (i, 0)),
            pl.BlockSpec((BM, 1), lambda i: (i, 0)),
            pl.BlockSpec((1, D), lambda i: (0, 0)),
        ],
        out_specs=pl.BlockSpec((BM, D), lambda i: (i, 0)),
        out_shape=jax.ShapeDtypeStruct((N, D), jnp.float32),
    )(a0, a1, hs, d2, b2)


def kernel(nodes_ft, adj_list, W, b):
    row = adj_list[0]
    col = adj_list[1]
    dis_pad = _dis_call(row)
    d2 = dis_pad[:N, None]
    hs = _matmul_call(nodes_ft, d2, W)
    pad_col = jnp.broadcast_to((jnp.arange(NPAD, dtype=jnp.int32) * 89) % N,
                               (NW, NPAD))
    pad_row = jnp.broadcast_to(
        N + jnp.arange(NPAD, dtype=jnp.int32) % (NPA - N), (NW, NPAD))
    row3 = jnp.concatenate([row.reshape(NW, EPC), pad_row],
                           axis=1).reshape(NW, NCH, CH)
    col3 = jnp.concatenate([col.reshape(NW, EPC), pad_col],
                           axis=1).reshape(NW, NCH, CH)
    acc = _scatter_call(hs, row3, col3)
    out = _merge_call(acc[0, :N], acc[1, :N], hs, d2, b.reshape(1, D))
    return out


# speedup vs baseline: 1.2645x; 1.0356x over previous
import functools

import jax
import jax.numpy as jnp
from jax import lax
from jax.experimental import pallas as pl
from jax.experimental.pallas import tpu as pltpu
from jax.experimental.pallas import tpu_sc as plsc

N = 10000
E = 320000
D = 128

NC = 2
NS = 16
L = 16
NW = NC * NS

NP = 10240
SPT = NP // NS
EPT_DEG = E // NS
NPA = 10112
RPT = NPA // NS
EPC = E // NW
CH = 120
NCH = 84
EPP = NCH * CH
NPAD = EPP - EPC

_MESH = plsc.VectorSubcoreMesh(
    core_axis_name="c", subcore_axis_name="s", num_cores=NC, num_subcores=NS)


def _rsqrt16(x):
    xi = plsc.bitcast(x, jnp.int32)
    yi = jnp.int32(0x5F3759DF) - (xi >> 1)
    y = plsc.bitcast(yi, jnp.float32)
    for _ in range(3):
        y = y * (1.5 - 0.5 * x * y * y)
    return y


def _dis_body(row_hbm, dis_hbm, idx_v, hist_v, dis_v, merged_v, shist):
    cid = lax.axis_index("c")
    sid = lax.axis_index("s")

    def zero(i, c):
        hist_v[pl.ds(i * L, L)] = jnp.zeros((L,), jnp.float32)
        return c
    lax.fori_loop(0, NP // L, zero, 0)

    pltpu.sync_copy(row_hbm.at[pl.ds(sid * EPT_DEG, EPT_DEG)], idx_v)

    ones = jnp.ones((L,), jnp.float32)

    def hist(i, c):
        for u in range(10):
            idx = idx_v[pl.ds(i * (10 * L) + u * L, L)]
            plsc.addupdate_scatter(hist_v, [idx], ones)
        return c
    lax.fori_loop(0, EPT_DEG // (10 * L), hist, 0)

    pltpu.sync_copy(hist_v, shist.at[sid])
    plsc.subcore_barrier()
    pltpu.sync_copy(shist.at[:, pl.ds(sid * SPT, SPT)], merged_v)

    def merge(j, c):
        acc = jnp.ones((L,), jnp.float32)
        for s in range(NS):
            acc = acc + merged_v[s, pl.ds(j * L, L)]
        dis_v[pl.ds(j * L, L)] = _rsqrt16(acc)
        return c
    lax.fori_loop(0, SPT // L, merge, 0)

    @pl.when(cid == 0)
    def _():
        pltpu.sync_copy(dis_v, dis_hbm.at[pl.ds(sid * SPT, SPT)])


_dis_call = functools.partial(
    pl.kernel,
    out_type=jax.ShapeDtypeStruct((NP,), jnp.float32),
    mesh=_MESH,
    scratch_types=[
        pltpu.VMEM((EPT_DEG,), jnp.int32),
        pltpu.VMEM((NP,), jnp.float32),
        pltpu.VMEM((SPT,), jnp.float32),
        pltpu.VMEM((NS, SPT), jnp.float32),
        pltpu.VMEM_SHARED((NS, NP), jnp.float32),
    ],
    compiler_params=pltpu.CompilerParams(needs_layout_passes=False),
)(_dis_body)


def _scatter_body(hs_hbm, row_hbm, col_hbm, acc_hbm,
                  idx_v, gbuf_v, acc_sh, csem, rsem, gsem):
    cid = lax.axis_index("c")
    sid = lax.axis_index("s")
    wid = cid * NS + sid

    def zrow(r, c):
        for u in range(D // L):
            gbuf_v[0, r, pl.ds(u * L, L)] = jnp.zeros((L,), jnp.float32)
        return c
    lax.fori_loop(0, CH, zrow, 0)
    for t in range(RPT // CH):
        pltpu.sync_copy(gbuf_v.at[0],
                        acc_sh.at[pl.ds(sid * RPT + t * CH, CH)])
    rem = RPT % CH
    if rem:
        pltpu.sync_copy(gbuf_v.at[0, pl.ds(0, rem)],
                        acc_sh.at[pl.ds(sid * RPT + RPT - rem, rem)])
    plsc.subcore_barrier()


    def load_idx(k):
        s = k % 4
        pltpu.async_copy(col_hbm.at[wid, k], idx_v.at[s], csem.at[s])
        pltpu.async_copy(row_hbm.at[wid, k], idx_v.at[4 + s], rsem.at[s])

    def wait_cidx(k):
        s = k % 4
        pltpu.make_async_copy(
            col_hbm.at[wid, k], idx_v.at[s], csem.at[s]).wait()

    def wait_ridx(k):
        s = k % 4
        pltpu.make_async_copy(
            row_hbm.at[wid, k], idx_v.at[4 + s], rsem.at[s]).wait()

    def gather(k):
        pltpu.async_copy(
            hs_hbm.at[idx_v.at[k % 4]], gbuf_v.at[k % 3], gsem.at[k % 3])

    def wait_gather(k):
        pltpu.make_async_copy(
            hs_hbm.at[idx_v.at[k % 4]], gbuf_v.at[k % 3],
            gsem.at[k % 3]).wait()

    load_idx(0)
    load_idx(1)
    load_idx(2)
    wait_cidx(0)
    gather(0)
    wait_cidx(1)
    gather(1)

    def body(k, c):
        @pl.when(k + 2 < NCH)
        def _():
            wait_cidx(k + 2)
            gather(k + 2)

        @pl.when(k + 3 < NCH)
        def _():
            load_idx(k + 3)

        wait_gather(k)
        wait_ridx(k)
        pltpu.sync_copy(gbuf_v.at[k % 3], acc_sh.at[idx_v.at[4 + k % 4]],
                        add=True)
        return c
    lax.fori_loop(0, NCH, body, 0)

    plsc.subcore_barrier()
    pltpu.sync_copy(acc_sh.at[pl.ds(sid * RPT, RPT)],
                    acc_hbm.at[cid, pl.ds(sid * RPT, RPT)])


_scatter_call = functools.partial(
    pl.kernel,
    out_type=jax.ShapeDtypeStruct((NC, NPA, D), jnp.float32),
    mesh=_MESH,
    scratch_types=[
        pltpu.VMEM((8, CH), jnp.int32),
        pltpu.VMEM((3, CH, D), jnp.float32),
        pltpu.VMEM_SHARED((NPA, D), jnp.float32),
        pltpu.SemaphoreType.DMA((4,)),
        pltpu.SemaphoreType.DMA((4,)),
        pltpu.SemaphoreType.DMA((3,)),
    ],
    compiler_params=pltpu.CompilerParams(needs_layout_passes=False),
)(_scatter_body)


BM = 400


def _matmul_body(x_ref, d_ref, w_ref, o_ref):
    o_ref[...] = jnp.dot(x_ref[...] * d_ref[...], w_ref[...],
                         preferred_element_type=jnp.float32)


def _matmul_call(x, d2, w):
    return pl.pallas_call(
        _matmul_body,
        grid=(N // BM,),
        in_specs=[
            pl.BlockSpec((BM, D), lambda i: (i, 0)),
            pl.BlockSpec((BM, 1), lambda i: (i, 0)),
            pl.BlockSpec((D, D), lambda i: (0, 0)),
        ],
        out_specs=pl.BlockSpec((BM, D), lambda i: (i, 0)),
        out_shape=jax.ShapeDtypeStruct((N, D), jnp.float32),
    )(x, d2, w)


def _merge_body(a0_ref, a1_ref, hs_ref, d_ref, b_ref, o_ref):
    o_ref[...] = (d_ref[...] * (a0_ref[0] + a1_ref[0] + hs_ref[...])
                  + b_ref[...])


def _merge_call(acc, hs, d2, b2):
    return pl.pallas_call(
        _merge_body,
        grid=(N // BM,),
        in_specs=[
            pl.BlockSpec((1, BM, D), lambda i: (0, i, 0)),
            pl.BlockSpec((1, BM, D), lambda i: (1, i, 0)),
            pl.BlockSpec((BM, D), lambda i: (i, 0)),
            pl.BlockSpec((BM, 1), lambda i: (i, 0)),
            pl.BlockSpec((1, D), lambda i: (0, 0)),
        ],
        out_specs=pl.BlockSpec((BM, D), lambda i: (i, 0)),
        out_shape=jax.ShapeDtypeStruct((N, D), jnp.float32),
    )(acc, acc, hs, d2, b2)


def kernel(nodes_ft, adj_list, W, b):
    row = adj_list[0]
    col = adj_list[1]
    dis_pad = _dis_call(row)
    d2 = dis_pad[:N, None]
    hs = _matmul_call(nodes_ft, d2, W)
    pad_col = jnp.broadcast_to((jnp.arange(NPAD, dtype=jnp.int32) * 89) % N,
                               (NW, NPAD))
    pad_row = jnp.broadcast_to(
        N + jnp.arange(NPAD, dtype=jnp.int32) % (NPA - N), (NW, NPAD))
    row3 = jnp.concatenate([row.reshape(NW, EPC), pad_row],
                           axis=1).reshape(NW, NCH, CH)
    col3 = jnp.concatenate([col.reshape(NW, EPC), pad_col],
                           axis=1).reshape(NW, NCH, CH)
    acc = _scatter_call(hs, row3, col3)
    out = _merge_call(acc, hs, d2, b.reshape(1, D))
    return out


# speedup vs baseline: 1.2780x; 1.0106x over previous
import functools

import jax
import jax.numpy as jnp
from jax import lax
from jax.experimental import pallas as pl
from jax.experimental.pallas import tpu as pltpu
from jax.experimental.pallas import tpu_sc as plsc

N = 10000
E = 320000
D = 128

NC = 2
NS = 16
L = 16
NW = NC * NS

NP = 10240
SPT = NP // NS
EPT_DEG = E // NS
NPA = 10112
RPT = NPA // NS
EPC = E // NW
CH = 120
NCH = 84
EPP = NCH * CH
NPAD = EPP - EPC

_MESH = plsc.VectorSubcoreMesh(
    core_axis_name="c", subcore_axis_name="s", num_cores=NC, num_subcores=NS)


def _rsqrt16(x):
    xi = plsc.bitcast(x, jnp.int32)
    yi = jnp.int32(0x5F3759DF) - (xi >> 1)
    y = plsc.bitcast(yi, jnp.float32)
    for _ in range(3):
        y = y * (1.5 - 0.5 * x * y * y)
    return y


def _dis_body(row_hbm, dis_hbm, idx_v, hist_v, dis_v, merged_v, shist):
    cid = lax.axis_index("c")
    sid = lax.axis_index("s")

    def zero(i, c):
        hist_v[pl.ds(i * L, L)] = jnp.zeros((L,), jnp.float32)
        return c
    lax.fori_loop(0, NP // L, zero, 0)

    pltpu.sync_copy(row_hbm.at[pl.ds(sid * EPT_DEG, EPT_DEG)], idx_v)

    ones = jnp.ones((L,), jnp.float32)

    def hist(i, c):
        for u in range(10):
            idx = idx_v[pl.ds(i * (10 * L) + u * L, L)]
            plsc.addupdate_scatter(hist_v, [idx], ones)
        return c
    lax.fori_loop(0, EPT_DEG // (10 * L), hist, 0)

    pltpu.sync_copy(hist_v, shist.at[sid])
    plsc.subcore_barrier()
    pltpu.sync_copy(shist.at[:, pl.ds(sid * SPT, SPT)], merged_v)

    def merge(j, c):
        acc = jnp.ones((L,), jnp.float32)
        for s in range(NS):
            acc = acc + merged_v[s, pl.ds(j * L, L)]
        dis_v[pl.ds(j * L, L)] = _rsqrt16(acc)
        return c
    lax.fori_loop(0, SPT // L, merge, 0)

    @pl.when(cid == 0)
    def _():
        pltpu.sync_copy(dis_v, dis_hbm.at[pl.ds(sid * SPT, SPT)])


_dis_call = functools.partial(
    pl.kernel,
    out_type=jax.ShapeDtypeStruct((NP,), jnp.float32),
    mesh=_MESH,
    scratch_types=[
        pltpu.VMEM((EPT_DEG,), jnp.int32),
        pltpu.VMEM((NP,), jnp.float32),
        pltpu.VMEM((SPT,), jnp.float32),
        pltpu.VMEM((NS, SPT), jnp.float32),
        pltpu.VMEM_SHARED((NS, NP), jnp.float32),
    ],
    compiler_params=pltpu.CompilerParams(needs_layout_passes=False),
)(_dis_body)


def _scatter_body(hs_hbm, row_hbm, col_hbm, acc_hbm,
                  idx_v, gbuf_v, acc_sh, csem, rsem, gsem):
    cid = lax.axis_index("c")
    sid = lax.axis_index("s")
    wid = cid * NS + sid

    def zrow(r, c):
        for u in range(D // L):
            gbuf_v[0, r, pl.ds(u * L, L)] = jnp.zeros((L,), jnp.float32)
        return c
    lax.fori_loop(0, CH, zrow, 0)
    for t in range(RPT // CH):
        pltpu.sync_copy(gbuf_v.at[0],
                        acc_sh.at[pl.ds(sid * RPT + t * CH, CH)])
    rem = RPT % CH
    if rem:
        pltpu.sync_copy(gbuf_v.at[0, pl.ds(0, rem)],
                        acc_sh.at[pl.ds(sid * RPT + RPT - rem, rem)])
    plsc.subcore_barrier()


    def load_idx(k):
        s = k % 4
        pltpu.async_copy(col_hbm.at[wid, k], idx_v.at[s], csem.at[s])
        pltpu.async_copy(row_hbm.at[wid, k], idx_v.at[4 + s], rsem.at[s])

    def wait_cidx(k):
        s = k % 4
        pltpu.make_async_copy(
            col_hbm.at[wid, k], idx_v.at[s], csem.at[s]).wait()

    def wait_ridx(k):
        s = k % 4
        pltpu.make_async_copy(
            row_hbm.at[wid, k], idx_v.at[4 + s], rsem.at[s]).wait()

    def gather(k):
        pltpu.async_copy(
            hs_hbm.at[idx_v.at[k % 4]], gbuf_v.at[k % 3], gsem.at[k % 3])

    def wait_gather(k):
        pltpu.make_async_copy(
            hs_hbm.at[idx_v.at[k % 4]], gbuf_v.at[k % 3],
            gsem.at[k % 3]).wait()

    load_idx(0)
    load_idx(1)
    load_idx(2)
    wait_cidx(0)
    gather(0)
    wait_cidx(1)
    gather(1)

    def body(k, c):
        @pl.when(k + 2 < NCH)
        def _():
            wait_cidx(k + 2)
            gather(k + 2)

        @pl.when(k + 3 < NCH)
        def _():
            load_idx(k + 3)

        wait_gather(k)
        wait_ridx(k)
        pltpu.sync_copy(gbuf_v.at[k % 3], acc_sh.at[idx_v.at[4 + k % 4]],
                        add=True)
        return c
    lax.fori_loop(0, NCH, body, 0)

    plsc.subcore_barrier()
    pltpu.sync_copy(acc_sh.at[pl.ds(sid * RPT, RPT)],
                    acc_hbm.at[cid, pl.ds(sid * RPT, RPT)])


_scatter_call = functools.partial(
    pl.kernel,
    out_type=jax.ShapeDtypeStruct((NC, NPA, D), jnp.float32),
    mesh=_MESH,
    scratch_types=[
        pltpu.VMEM((8, CH), jnp.int32),
        pltpu.VMEM((3, CH, D), jnp.float32),
        pltpu.VMEM_SHARED((NPA, D), jnp.float32),
        pltpu.SemaphoreType.DMA((4,)),
        pltpu.SemaphoreType.DMA((4,)),
        pltpu.SemaphoreType.DMA((3,)),
    ],
    compiler_params=pltpu.CompilerParams(needs_layout_passes=False),
)(_scatter_body)


BM = 400


def _matmul_body(x_ref, d_ref, w_ref, o_ref):
    o_ref[...] = jnp.dot(x_ref[...] * d_ref[...], w_ref[...],
                         preferred_element_type=jnp.float32)


def _matmul_call(x, d2, w):
    return pl.pallas_call(
        _matmul_body,
        grid=(N // BM,),
        in_specs=[
            pl.BlockSpec((BM, D), lambda i: (i, 0)),
            pl.BlockSpec((BM, 1), lambda i: (i, 0)),
            pl.BlockSpec((D, D), lambda i: (0, 0)),
        ],
        out_specs=pl.BlockSpec((BM, D), lambda i: (i, 0)),
        out_shape=jax.ShapeDtypeStruct((N, D), jnp.float32),
    )(x, d2, w)


def _merge_body(a0_ref, a1_ref, hs_ref, d_ref, b_ref, o_ref):
    o_ref[...] = (d_ref[...] * (a0_ref[0] + a1_ref[0] + hs_ref[...])
                  + b_ref[...])


def _merge_call(acc, hs, d2, b2):
    return pl.pallas_call(
        _merge_body,
        grid=(N // BM,),
        in_specs=[
            pl.BlockSpec((1, BM, D), lambda i: (0, i, 0)),
            pl.BlockSpec((1, BM, D), lambda i: (1, i, 0)),
            pl.BlockSpec((BM, D), lambda i: (i, 0)),
            pl.BlockSpec((BM, 1), lambda i: (i, 0)),
            pl.BlockSpec((1, D), lambda i: (0, 0)),
        ],
        out_specs=pl.BlockSpec((BM, D), lambda i: (i, 0)),
        out_shape=jax.ShapeDtypeStruct((N, D), jnp.float32),
    )(acc, acc, hs, d2, b2)


def kernel(nodes_ft, adj_list, W, b):
    row = adj_list[0]
    col = adj_list[1]
    dis_pad = _dis_call(row)
    d2 = dis_pad[:, None]
    hs = _matmul_call(nodes_ft, d2, W)
    pad_col = jnp.broadcast_to((jnp.arange(NPAD, dtype=jnp.int32) * 89) % N,
                               (NW, NPAD))
    pad_row = jnp.broadcast_to(
        N + jnp.arange(NPAD, dtype=jnp.int32) % (NPA - N), (NW, NPAD))
    row3 = jnp.concatenate([row.reshape(NW, EPC), pad_row],
                           axis=1).reshape(NW, NCH, CH)
    col3 = jnp.concatenate([col.reshape(NW, EPC), pad_col],
                           axis=1).reshape(NW, NCH, CH)
    acc = _scatter_call(hs, row3, col3)
    out = _merge_call(acc, hs, d2, b.reshape(1, D))
    return out
